# baseline (speedup 1.0000x reference)
import functools

import jax
import jax.numpy as jnp
from jax import lax
from jax.experimental import pallas as pl
from jax.experimental.pallas import tpu as pltpu
from jax.experimental.pallas import tpu_sc as plsc

H = 2048
E = 16
LW = 128
NC = 2
NS = 16
NW = NC * NS
L = 16


def _logits_body(h_ref, w_ref, b_ref, out_ref):
    tm = h_ref.shape[0]
    acc = lax.dot_general(h_ref[...], w_ref[...],
                          dimension_numbers=(((1,), (1,)), ((), ())),
                          preferred_element_type=jnp.float32)
    pad = jnp.zeros((tm, LW - E), jnp.float32)
    out_ref[...] = jnp.concatenate([acc + b_ref[...], pad], axis=1)


def _compute_logits(h2, W, bias_row):
    m = h2.shape[0]
    tm = 1024
    return pl.pallas_call(
        _logits_body,
        grid=(m // tm,),
        in_specs=[
            pl.BlockSpec((tm, H), lambda i: (i, 0)),
            pl.BlockSpec((E, H), lambda i: (0, 0)),
            pl.BlockSpec((1, E), lambda i: (0, 0)),
        ],
        out_specs=pl.BlockSpec((tm, LW), lambda i: (i, 0)),
        out_shape=jax.ShapeDtypeStruct((m, LW), jnp.float32),
    )(h2, W, bias_row)


def _make_router(tok):
    chunk = tok // NW
    groups = chunk // L
    mesh = plsc.VectorSubcoreMesh(core_axis_name="c", subcore_axis_name="s")

    @functools.partial(
        pl.kernel,
        mesh=mesh,
        out_type=(
            jax.ShapeDtypeStruct((tok, LW), jnp.float32),
            jax.ShapeDtypeStruct((NW * E,), jnp.float32),
            jax.ShapeDtypeStruct((NW * E,), jnp.float32),
        ),
        scratch_types=[
            pltpu.VMEM((chunk, E), jnp.float32),
            pltpu.VMEM((chunk, E), jnp.float32),
            pltpu.VMEM((E,), jnp.float32),
        ],
        compiler_params=pltpu.CompilerParams(
            needs_layout_passes=False, use_tc_tiling_on_sc=False,
            disable_bounds_checks=True, skip_device_barrier=True),
    )
    def router(logits_hbm, route_hbm, imp_hbm, load_hbm, lv, rv, pv):
        wid = lax.axis_index("s") * NC + lax.axis_index("c")
        base = wid * chunk
        pltpu.sync_copy(logits_hbm.at[pl.ds(base, chunk), pl.ds(0, E)], lv)

        lanes = lax.iota(jnp.int32, L)
        zero = jnp.zeros((L,), jnp.float32)
        ninf = jnp.full((L,), -jnp.inf, jnp.float32)

        def group(g, carry):
            imp, ld = carry
            tok_idx = g * L + lanes
            r = [plsc.load_gather(lv, [tok_idx, jnp.full((L,), e, jnp.int32)])
                 for e in range(E)]
            m1 = r[0]
            for e in range(1, E):
                m1 = jnp.maximum(m1, r[e])
            first = jnp.full((L,), E, jnp.int32)
            for e in range(E):
                hit = (r[e] == m1) & (first == E)
                first = jnp.where(hit, e, first)
            sumex = zero
            for e in range(E):
                sumex = sumex + jnp.exp(r[e] - m1)
            inv_sumex = 1.0 / sumex
            m2 = ninf
            for e in range(E):
                m2 = jnp.maximum(m2, jnp.where(first == e, ninf, r[e]))
            em = jnp.exp(m2 - m1)
            invd = 1.0 / (1.0 + em)
            s1 = invd
            s2 = em * invd
            second = jnp.full((L,), E, jnp.int32)
            new_imp, new_ld = [], []
            for e in range(E):
                new_imp.append(imp[e] + jnp.exp(r[e] - m1) * inv_sumex)
                isf = first == e
                hit2 = (r[e] == m2) & (second == E) & (first != e)
                second = jnp.where(hit2, e, second)
                route_e = jnp.where(isf, s1, jnp.where(hit2, s2, 0.0))
                new_ld.append(ld[e] + jnp.where(route_e > 0, 1.0, 0.0))
                plsc.store_scatter(rv, [tok_idx, jnp.full((L,), e, jnp.int32)],
                                   route_e)
            return tuple(new_imp), tuple(new_ld)

        imp, ld = lax.fori_loop(0, groups, group,
                                (tuple(zero for _ in range(E)),
                                 tuple(zero for _ in range(E))))
        pltpu.sync_copy(rv, route_hbm.at[pl.ds(base, chunk), pl.ds(0, E)])

        impv = zero
        for e in range(E):
            impv = jnp.where(lanes == e, jnp.sum(imp[e]), impv)
        pv[...] = impv
        pltpu.sync_copy(pv, imp_hbm.at[pl.ds(wid * E, E)])
        ldv = zero
        for e in range(E):
            ldv = jnp.where(lanes == e, jnp.sum(ld[e]), ldv)
        pv[...] = ldv
        pltpu.sync_copy(pv, load_hbm.at[pl.ds(wid * E, E)])

    return router


def kernel(h, W, bias):
    b, s, _ = h.shape
    tok = b * s
    h2 = h.reshape(tok, H)
    logits = _compute_logits(h2, W, bias.reshape(1, E))
    route_w, imp_part, load_part = _make_router(tok)(logits)
    importance = imp_part.reshape(NW, E).sum(axis=0)
    load = load_part.reshape(NW, E).sum(axis=0) / tok
    aux_loss = jnp.mean(importance * load * (E * E))
    return route_w[:, :E].reshape(b, s, E), aux_loss


# speedup vs baseline: 1.0113x; 1.0113x over previous
import functools

import jax
import jax.numpy as jnp
from jax import lax
from jax.experimental import pallas as pl
from jax.experimental.pallas import tpu as pltpu
from jax.experimental.pallas import tpu_sc as plsc

H = 2048
E = 16
LW = 128
NC = 2
NS = 16
NW = NC * NS
L = 16


def _logits_body(h_ref, w_ref, b_ref, out_ref):
    tm = h_ref.shape[0]
    acc = lax.dot_general(h_ref[...], w_ref[...],
                          dimension_numbers=(((1,), (1,)), ((), ())),
                          preferred_element_type=jnp.float32)
    pad = jnp.zeros((tm, LW - E), jnp.float32)
    out_ref[...] = jnp.concatenate([acc + b_ref[...], pad], axis=1)


def _compute_logits(h2, W, bias_row):
    m = h2.shape[0]
    tm = 1024
    return pl.pallas_call(
        _logits_body,
        grid=(m // tm,),
        in_specs=[
            pl.BlockSpec((tm, H), lambda i: (i, 0)),
            pl.BlockSpec((E, H), lambda i: (0, 0)),
            pl.BlockSpec((1, E), lambda i: (0, 0)),
        ],
        out_specs=pl.BlockSpec((tm, LW), lambda i: (i, 0)),
        out_shape=jax.ShapeDtypeStruct((m, LW), jnp.float32),
    )(h2, W, bias_row)


def _make_router(tok):
    chunk = tok // NW
    groups = chunk // L
    mesh = plsc.VectorSubcoreMesh(core_axis_name="c", subcore_axis_name="s")

    @functools.partial(
        pl.kernel,
        mesh=mesh,
        out_type=(
            jax.ShapeDtypeStruct((tok, LW), jnp.float32),
            jax.ShapeDtypeStruct((NW * E,), jnp.float32),
            jax.ShapeDtypeStruct((NW * E,), jnp.float32),
        ),
        scratch_types=[
            pltpu.VMEM((chunk, E), jnp.float32),
            pltpu.VMEM((chunk, E), jnp.float32),
            pltpu.VMEM((E,), jnp.float32),
        ],
        compiler_params=pltpu.CompilerParams(
            needs_layout_passes=False, use_tc_tiling_on_sc=False),
    )
    def router(logits_hbm, route_hbm, imp_hbm, load_hbm, lv, rv, pv):
        wid = lax.axis_index("s") * NC + lax.axis_index("c")
        base = wid * chunk
        pltpu.sync_copy(logits_hbm.at[pl.ds(base, chunk), pl.ds(0, E)], lv)

        lanes = lax.iota(jnp.int32, L)
        zero = jnp.zeros((L,), jnp.float32)
        ninf = jnp.full((L,), -jnp.inf, jnp.float32)

        def group(g, carry):
            imp, ld = carry
            tok_idx = g * L + lanes
            r = [plsc.load_gather(lv, [tok_idx, jnp.full((L,), e, jnp.int32)])
                 for e in range(E)]
            m1 = r[0]
            i1 = jnp.zeros((L,), jnp.int32)
            m2 = ninf
            i2 = jnp.full((L,), E, jnp.int32)
            for e in range(1, E):
                gt1 = r[e] > m1
                gt2 = (~gt1) & (r[e] > m2)
                m2 = jnp.where(gt1, m1, jnp.where(gt2, r[e], m2))
                i2 = jnp.where(gt1, i1, jnp.where(gt2, e, i2))
                m1 = jnp.where(gt1, r[e], m1)
                i1 = jnp.where(gt1, e, i1)
            sumex = zero
            for e in range(E):
                sumex = sumex + jnp.exp(r[e] - m1)
            inv_sumex = 1.0 / sumex
            em = jnp.exp(m2 - m1)
            invd = 1.0 / (1.0 + em)
            s1 = invd
            s2 = em * invd
            s2pos = s2 > 0
            new_imp, new_ld = [], []
            for e in range(E):
                new_imp.append(imp[e] + jnp.exp(r[e] - m1) * inv_sumex)
                on1 = i1 == e
                on2 = i2 == e
                route_e = jnp.where(on1, s1, jnp.where(on2, s2, 0.0))
                new_ld.append(ld[e] + jnp.where(on1 | (on2 & s2pos), 1.0, 0.0))
                plsc.store_scatter(rv, [tok_idx, jnp.full((L,), e, jnp.int32)],
                                   route_e)
            return tuple(new_imp), tuple(new_ld)

        imp, ld = lax.fori_loop(0, groups, group,
                                (tuple(zero for _ in range(E)),
                                 tuple(zero for _ in range(E))))
        pltpu.sync_copy(rv, route_hbm.at[pl.ds(base, chunk), pl.ds(0, E)])

        impv = zero
        for e in range(E):
            impv = jnp.where(lanes == e, jnp.sum(imp[e]), impv)
        pv[...] = impv
        pltpu.sync_copy(pv, imp_hbm.at[pl.ds(wid * E, E)])
        ldv = zero
        for e in range(E):
            ldv = jnp.where(lanes == e, jnp.sum(ld[e]), ldv)
        pv[...] = ldv
        pltpu.sync_copy(pv, load_hbm.at[pl.ds(wid * E, E)])

    return router


def kernel(h, W, bias):
    b, s, _ = h.shape
    tok = b * s
    h2 = h.reshape(tok, H)
    logits = _compute_logits(h2, W, bias.reshape(1, E))
    route_w, imp_part, load_part = _make_router(tok)(logits)
    importance = imp_part.reshape(NW, E).sum(axis=0)
    load = load_part.reshape(NW, E).sum(axis=0) / tok
    aux_loss = jnp.mean(importance * load * (E * E))
    return route_w[:, :E].reshape(b, s, E), aux_loss
